# baseline (speedup 1.0000x reference)
import functools

import jax
import jax.numpy as jnp
from jax import lax
from jax.experimental import pallas as pl
from jax.experimental.pallas import tpu as pltpu
from jax.experimental.pallas import tpu_sc as plsc

V = 1000000
D = 64
B = 4096
S = 200
C = 100
CPAD = 128

NC, NS = 2, 16
NW = NC * NS
BPW = B // NW
CHUNK = 100
NSLOT = S // CHUNK
GPW = BPW * NSLOT
NEG = float("-inf")

VB = 16384


def _tc_pack(table_t):
    def tr(ta_ref, tb_ref, o_ref):
        tcat = jnp.concatenate([ta_ref[...], tb_ref[...]], axis=0)
        o_ref[...] = jnp.swapaxes(tcat, 0, 1)

    nspan = pl.cdiv(V, 2 * VB)
    nblk = pl.cdiv(V, VB) - 1
    return pl.pallas_call(
        tr,
        grid=(nspan,),
        in_specs=[
            pl.BlockSpec((D, VB), lambda i: (0, 2 * i)),
            pl.BlockSpec((D, VB), lambda i: (0, jnp.minimum(2 * i + 1, nblk))),
        ],
        out_specs=pl.BlockSpec((VB, 128), lambda i: (i, 0)),
        out_shape=jax.ShapeDtypeStruct((nspan * VB, 128), jnp.float32),
    )(table_t, table_t)


def _sc_gather_maxpool(x2, table_lin):
    mesh = plsc.VectorSubcoreMesh(core_axis_name="c", subcore_axis_name="s")

    @functools.partial(
        pl.kernel,
        out_type=jax.ShapeDtypeStruct((B, D), jnp.float32),
        mesh=mesh,
        compiler_params=pltpu.CompilerParams(use_tc_tiling_on_sc=False),
        scratch_types=[
            pltpu.VMEM((GPW, CHUNK), jnp.int32),
            pltpu.VMEM((CHUNK, D), jnp.float32),
            pltpu.VMEM((CHUNK, D), jnp.float32),
            pltpu.VMEM((CHUNK, D), jnp.float32),
            pltpu.VMEM((CHUNK, D), jnp.float32),
            pltpu.VMEM((BPW, D), jnp.float32),
            pltpu.SemaphoreType.DMA,
            pltpu.SemaphoreType.DMA,
            pltpu.SemaphoreType.DMA,
            pltpu.SemaphoreType.DMA,
        ],
    )
    def k(x_hbm, table_hbm, out_hbm, idx_v, buf0, buf1, buf2, buf3, out_v,
          sem0, sem1, sem2, sem3):
        wid = lax.axis_index("s") * NC + lax.axis_index("c")
        pltpu.sync_copy(x_hbm.at[pl.ds(wid * GPW, GPW)], idx_v)
        bufs = (buf0, buf1, buf2, buf3)
        sems = (sem0, sem1, sem2, sem3)
        for r in range(4):
            pltpu.async_copy(table_hbm.at[idx_v.at[r]], bufs[r], sems[r])

        def pool_pair(pp, carry):
            for half in range(2):
                p = 2 * pp + half
                accs = tuple(jnp.full((16,), NEG, jnp.float32) for _ in range(4))
                for slot in range(NSLOT):
                    g = 4 * pp + NSLOT * half + slot
                    ring = 2 * half + slot
                    buf = bufs[ring]
                    sem = sems[ring]
                    pltpu.make_async_copy(table_hbm.at[idx_v.at[g]], buf, sem).wait()

                    def step(jj, a, buf=buf):
                        res = list(a)
                        for u in range(20):
                            j = jj * 20 + u
                            for q in range(4):
                                res[q] = jnp.maximum(
                                    res[q], buf[j, pl.ds(16 * q, 16)])
                        return tuple(res)

                    accs = lax.fori_loop(0, CHUNK // 20, step, accs)

                    @pl.when(pp < BPW // 2 - 1)
                    def _(g=g, buf=buf, sem=sem):
                        pltpu.async_copy(table_hbm.at[idx_v.at[g + 4]], buf, sem)

                for q in range(4):
                    out_v[p, pl.ds(16 * q, 16)] = accs[q]
            return carry

        lax.fori_loop(0, BPW // 2, pool_pair, 0)
        pltpu.sync_copy(out_v, out_hbm.at[pl.ds(wid * BPW, BPW)])

    return k(x2, table_lin)


def _tc_linear(pooled, w_pad, b_pad):
    def mm(p_ref, w_ref, b_ref, o_ref):
        o_ref[...] = (
            lax.dot_general(
                p_ref[...], w_ref[...], (((1,), (1,)), ((), ())),
                preferred_element_type=jnp.float32,
            )
            + b_ref[...]
        )

    return pl.pallas_call(
        mm,
        out_shape=jax.ShapeDtypeStruct((B, CPAD), jnp.float32),
    )(pooled, w_pad, b_pad)


def kernel(x, table, W, b):
    x = x.astype(jnp.int32)
    hs = VB.bit_length() - 1
    sp = x >> (hs + 1)
    w = x & (2 * VB - 1)
    x2 = ((sp << (hs + 1)) + ((w & (VB - 1)) << 1) + (w >> hs)).reshape(
        B * S // CHUNK, CHUNK)
    table_lin = _tc_pack(table.T).reshape(-1, D)
    pooled = _sc_gather_maxpool(x2, table_lin)
    w_pad = jnp.zeros((CPAD, D), jnp.float32).at[:C].set(W)
    b_pad = jnp.zeros((1, CPAD), jnp.float32).at[0, :C].set(b)
    return _tc_linear(pooled, w_pad, b_pad)[:, :C]


# speedup vs baseline: 1.0124x; 1.0124x over previous
import functools

import jax
import jax.numpy as jnp
from jax import lax
from jax.experimental import pallas as pl
from jax.experimental.pallas import tpu as pltpu
from jax.experimental.pallas import tpu_sc as plsc

V = 1000000
D = 64
B = 4096
S = 200
C = 100
CPAD = 128

NC, NS = 2, 16
NW = NC * NS
BPW = B // NW
CHUNK = 100
NSLOT = S // CHUNK
GPW = BPW * NSLOT
NEG = float("-inf")

VB = 16384


def _tc_pack(table_t):
    def tr(ta_ref, tb_ref, o_ref):
        tcat = jnp.concatenate([ta_ref[...], tb_ref[...]], axis=0)
        o_ref[...] = jnp.swapaxes(tcat, 0, 1)

    nspan = pl.cdiv(V, 2 * VB)
    nblk = pl.cdiv(V, VB) - 1
    return pl.pallas_call(
        tr,
        grid=(nspan,),
        in_specs=[
            pl.BlockSpec((D, VB), lambda i: (0, 2 * i)),
            pl.BlockSpec((D, VB), lambda i: (0, jnp.minimum(2 * i + 1, nblk))),
        ],
        out_specs=pl.BlockSpec((VB, 128), lambda i: (i, 0)),
        out_shape=jax.ShapeDtypeStruct((nspan * VB, 128), jnp.float32),
    )(table_t, table_t)


def _sc_gather_maxpool(x2, table_lin):
    mesh = plsc.VectorSubcoreMesh(core_axis_name="c", subcore_axis_name="s")

    @functools.partial(
        pl.kernel,
        out_type=jax.ShapeDtypeStruct((B, D), jnp.float32),
        mesh=mesh,
        compiler_params=pltpu.CompilerParams(use_tc_tiling_on_sc=False),
        scratch_types=[
            pltpu.VMEM((GPW, CHUNK), jnp.int32),
            pltpu.VMEM((CHUNK, D), jnp.float32),
            pltpu.VMEM((CHUNK, D), jnp.float32),
            pltpu.VMEM((CHUNK, D), jnp.float32),
            pltpu.VMEM((CHUNK, D), jnp.float32),
            pltpu.VMEM((BPW, D), jnp.float32),
            pltpu.SemaphoreType.DMA,
            pltpu.SemaphoreType.DMA,
            pltpu.SemaphoreType.DMA,
            pltpu.SemaphoreType.DMA,
        ],
    )
    def k(x_hbm, table_hbm, out_hbm, idx_v, buf0, buf1, buf2, buf3, out_v,
          sem0, sem1, sem2, sem3):
        wid = lax.axis_index("s") * NC + lax.axis_index("c")
        pltpu.sync_copy(x_hbm.at[pl.ds(wid * GPW, GPW)], idx_v)
        bufs = (buf0, buf1, buf2, buf3)
        sems = (sem0, sem1, sem2, sem3)
        for r in range(4):
            pltpu.async_copy(table_hbm.at[idx_v.at[r]], bufs[r], sems[r])

        def pool_pair(pp, carry):
            for half in range(2):
                p = 2 * pp + half
                accs = tuple(jnp.full((16,), NEG, jnp.float32) for _ in range(4))
                for slot in range(NSLOT):
                    g = 4 * pp + NSLOT * half + slot
                    ring = 2 * half + slot
                    buf = bufs[ring]
                    sem = sems[ring]
                    pltpu.make_async_copy(table_hbm.at[idx_v.at[g]], buf, sem).wait()

                    def step(jj, a, buf=buf):
                        res = list(a)
                        for u in range(10):
                            j = jj * 10 + u
                            for q in range(4):
                                res[q] = jnp.maximum(
                                    res[q], buf[j, pl.ds(16 * q, 16)])
                        return tuple(res)

                    accs = lax.fori_loop(0, CHUNK // 10, step, accs)

                    @pl.when(pp < BPW // 2 - 1)
                    def _(g=g, buf=buf, sem=sem):
                        pltpu.async_copy(table_hbm.at[idx_v.at[g + 4]], buf, sem)

                for q in range(4):
                    out_v[p, pl.ds(16 * q, 16)] = accs[q]
            return carry

        lax.fori_loop(0, BPW // 2, pool_pair, 0)
        pltpu.sync_copy(out_v, out_hbm.at[pl.ds(wid * BPW, BPW)])

    return k(x2, table_lin)


def _tc_linear(pooled, w_pad, b_pad):
    def mm(p_ref, w_ref, b_ref, o_ref):
        o_ref[...] = (
            lax.dot_general(
                p_ref[...], w_ref[...], (((1,), (1,)), ((), ())),
                preferred_element_type=jnp.float32,
            )
            + b_ref[...]
        )

    return pl.pallas_call(
        mm,
        out_shape=jax.ShapeDtypeStruct((B, CPAD), jnp.float32),
    )(pooled, w_pad, b_pad)


def kernel(x, table, W, b):
    x = x.astype(jnp.int32)
    hs = VB.bit_length() - 1
    sp = x >> (hs + 1)
    w = x & (2 * VB - 1)
    x2 = ((sp << (hs + 1)) + ((w & (VB - 1)) << 1) + (w >> hs)).reshape(
        B * S // CHUNK, CHUNK)
    table_lin = _tc_pack(table.T).reshape(-1, D)
    pooled = _sc_gather_maxpool(x2, table_lin)
    w_pad = jnp.zeros((CPAD, D), jnp.float32).at[:C].set(W)
    b_pad = jnp.zeros((1, CPAD), jnp.float32).at[0, :C].set(b)
    return _tc_linear(pooled, w_pad, b_pad)[:, :C]
